# baseline (speedup 1.0000x reference)
import functools

import jax
import jax.numpy as jnp
from jax import lax
from jax.experimental import pallas as pl
from jax.experimental.pallas import tpu as pltpu
from jax.experimental.pallas import tpu_sc as plsc

N_IN, N_OUT, K, C_IN, C_OUT, HIDDEN, BATCH = 10000, 1000, 16, 128, 32, 32, 4

_NW = 32
_ROWS = N_OUT * BATCH * K
_ROWS_PAD = 65536
_ROWS_PER_W = _ROWS_PAD // _NW
_CHUNK = 128
_NCHUNK = _ROWS_PER_W // _CHUNK


def _sc_gather(table, idx):
    mesh = plsc.VectorSubcoreMesh(core_axis_name="c", subcore_axis_name="s")

    @functools.partial(
        pl.kernel,
        mesh=mesh,
        out_type=jax.ShapeDtypeStruct((_ROWS_PAD, C_IN), jnp.float32),
        scratch_types=[
            pltpu.VMEM((_CHUNK,), jnp.int32),
            pltpu.VMEM((_CHUNK, C_IN), jnp.float32),
            pltpu.SemaphoreType.DMA,
        ],
    )
    def k(table_hbm, idx_hbm, out_hbm, idx_v, rows_v, sem):
        wid = lax.axis_index("s") * 2 + lax.axis_index("c")
        base = wid * _ROWS_PER_W

        def body(g, carry):
            off = base + g * _CHUNK
            pltpu.sync_copy(idx_hbm.at[pl.ds(off, _CHUNK)], idx_v)
            pltpu.async_copy(table_hbm.at[idx_v], rows_v, sem).wait()
            pltpu.sync_copy(rows_v, out_hbm.at[pl.ds(off, _CHUNK)])
            return carry

        lax.fori_loop(0, _NCHUNK, body, 0)

    return k(table, idx)


_LB = 40


def _tc_body(cx_ref, cy_ref, xg_ref, wp_ref, w2q_ref, b2r_ref, out_ref):
    f32, bf16 = jnp.float32, jnp.bfloat16
    cx = cx_ref[...].astype(bf16).astype(f32)
    cy = cy_ref[...].astype(bf16).astype(f32)
    w1x = wp_ref[0, :].astype(bf16).astype(f32)
    w1y = wp_ref[1, :].astype(bf16).astype(f32)
    arg = (cx[:, :, None] * w1x[None, None, :]
           + cy[:, :, None] * w1y[None, None, :]) + wp_ref[2, :][None, None, :]
    h = jnp.sin(30.0 * arg)
    xg = xg_ref[...]
    acc = jnp.zeros((_LB, BATCH, HIDDEN, C_IN), jnp.float32)
    for k in range(K):
        acc = acc + (h[:, k, :][:, None, :, None]
                     * xg[:, :, k, :][:, :, None, :])
    p = acc.reshape(_LB * BATCH, HIDDEN * C_IN)
    s = jnp.sum(xg, axis=2).reshape(_LB * BATCH, C_IN)
    res = (jnp.dot(p, w2q_ref[...], preferred_element_type=jnp.float32,
                   precision=lax.Precision.HIGHEST)
           + jnp.dot(s, b2r_ref[...], preferred_element_type=jnp.float32,
                     precision=lax.Precision.HIGHEST)
           + wp_ref[3, :][None, :])
    out_ref[...] = res


def _tc_call(cx, cy, xg4, wp, w2q, b2r):
    grid = (N_OUT // _LB,)
    return pl.pallas_call(
        _tc_body,
        grid=grid,
        in_specs=[
            pl.BlockSpec((_LB, K), lambda i: (i, 0)),
            pl.BlockSpec((_LB, K), lambda i: (i, 0)),
            pl.BlockSpec((_LB, BATCH, K, C_IN), lambda i: (i, 0, 0, 0)),
            pl.BlockSpec((8, HIDDEN), lambda i: (0, 0)),
            pl.BlockSpec((HIDDEN * C_IN, C_OUT), lambda i: (0, 0)),
            pl.BlockSpec((C_IN, C_OUT), lambda i: (0, 0)),
        ],
        out_specs=pl.BlockSpec((_LB * BATCH, C_OUT), lambda i: (i, 0)),
        out_shape=jax.ShapeDtypeStruct((N_OUT * BATCH, C_OUT), jnp.float32),
    )(cx, cy, xg4, wp, w2q, b2r)


def kernel(x, locs_unfold, W1, b1, W2, b2, bias, neighbours):
    lu3 = locs_unfold.reshape(2, K, N_OUT)
    cx = lu3[0].T
    cy = lu3[1].T
    wp = jnp.zeros((8, HIDDEN), jnp.float32)
    wp = wp.at[0].set(W1[0]).at[1].set(W1[1]).at[2].set(b1)
    wp = wp.at[3].set(bias[0, :, 0])
    w2q = W2.reshape(HIDDEN, C_OUT, C_IN).transpose(0, 2, 1).reshape(
        HIDDEN * C_IN, C_OUT)
    b2r = b2.reshape(C_OUT, C_IN).T

    table = x.transpose(0, 2, 1).reshape(BATCH * N_IN, C_IN)
    idx = (neighbours.T[:, None, :]
           + (jnp.arange(BATCH, dtype=jnp.int32) * N_IN)[None, :, None])
    idx_pad = jnp.zeros((_ROWS_PAD,), jnp.int32).at[:_ROWS].set(idx.reshape(_ROWS))
    xg = _sc_gather(table, idx_pad)
    xg4 = xg[:_ROWS].reshape(N_OUT, BATCH, K, C_IN)

    out2 = _tc_call(cx, cy, xg4, wp, w2q, b2r)
    return out2.reshape(N_OUT, BATCH, C_OUT).transpose(1, 2, 0)


# speedup vs baseline: 1.1811x; 1.1811x over previous
import functools

import jax
import jax.numpy as jnp
from jax import lax
from jax.experimental import pallas as pl
from jax.experimental.pallas import tpu as pltpu
from jax.experimental.pallas import tpu_sc as plsc

N_IN, N_OUT, K, C_IN, C_OUT, HIDDEN, BATCH = 10000, 1000, 16, 128, 32, 32, 4

_NW = 32
_D = BATCH * C_IN
_ROWS = N_OUT * K
_ROWS_PAD = 16384
_ROWS_PER_W = _ROWS_PAD // _NW
_CHUNK = 64
_NCHUNK = _ROWS_PER_W // _CHUNK


def _sc_gather(table, idx3):
    mesh = plsc.VectorSubcoreMesh(core_axis_name="c", subcore_axis_name="s")

    @functools.partial(
        pl.kernel,
        mesh=mesh,
        out_type=jax.ShapeDtypeStruct((_ROWS_PAD, _D), jnp.float32),
        scratch_types=[
            pltpu.VMEM((_NCHUNK, _CHUNK), jnp.int32),
            pltpu.VMEM((_CHUNK, _D), jnp.float32),
            pltpu.VMEM((_CHUNK, _D), jnp.float32),
            pltpu.SemaphoreType.DMA,
            pltpu.SemaphoreType.DMA,
            pltpu.SemaphoreType.DMA,
            pltpu.SemaphoreType.DMA,
        ],
    )
    def k(table_hbm, idx_hbm, out_hbm, idx_v, rows0, rows1, sg0, sg1, sw0, sw1):
        wid = lax.axis_index("s") * 2 + lax.axis_index("c")
        base = wid * _ROWS_PER_W
        pltpu.sync_copy(idx_hbm.at[wid], idx_v)
        bufs, gsems, wsems = (rows0, rows1), (sg0, sg1), (sw0, sw1)

        gh = [pltpu.async_copy(table_hbm.at[idx_v.at[g]], bufs[g], gsems[g])
              for g in range(2)]
        wh = [None, None]
        for g in range(_NCHUNK):
            b = g & 1
            gh[b].wait()
            wh[b] = pltpu.async_copy(
                bufs[b], out_hbm.at[pl.ds(base + g * _CHUNK, _CHUNK)], wsems[b])
            if g + 2 < _NCHUNK:
                wh[b].wait()
                gh[b] = pltpu.async_copy(
                    table_hbm.at[idx_v.at[g + 2]], bufs[b], gsems[b])
        wh[0].wait()
        wh[1].wait()

    return k(table, idx3)


_LB = 40


def _tc_body(cx_ref, cy_ref, xg_ref, wp_ref, w2q_ref, b2r_ref, out_ref):
    f32, bf16 = jnp.float32, jnp.bfloat16
    cx = cx_ref[...].astype(bf16).astype(f32)
    cy = cy_ref[...].astype(bf16).astype(f32)
    w1x = wp_ref[0, :].astype(bf16).astype(f32)
    w1y = wp_ref[1, :].astype(bf16).astype(f32)
    arg = (cx[:, :, None] * w1x[None, None, :]
           + cy[:, :, None] * w1y[None, None, :]) + wp_ref[2, :][None, None, :]
    h = jnp.sin(30.0 * arg)
    xg = xg_ref[...]
    acc = jnp.zeros((_LB, BATCH, HIDDEN, C_IN), jnp.float32)
    for k in range(K):
        acc = acc + (h[:, k, :][:, None, :, None]
                     * xg[:, k, :, :][:, :, None, :])
    p = acc.reshape(_LB * BATCH, HIDDEN * C_IN)
    s = jnp.sum(xg, axis=1).reshape(_LB * BATCH, C_IN)
    res = (jnp.dot(p, w2q_ref[...], preferred_element_type=jnp.float32,
                   precision=lax.Precision.HIGHEST)
           + jnp.dot(s, b2r_ref[...], preferred_element_type=jnp.float32,
                     precision=lax.Precision.HIGHEST)
           + wp_ref[3, :][None, :])
    out_ref[...] = res


def _tc_call(cx, cy, xg4, wp, w2q, b2r):
    grid = (N_OUT // _LB,)
    return pl.pallas_call(
        _tc_body,
        grid=grid,
        in_specs=[
            pl.BlockSpec((_LB, K), lambda i: (i, 0)),
            pl.BlockSpec((_LB, K), lambda i: (i, 0)),
            pl.BlockSpec((_LB, K, BATCH, C_IN), lambda i: (i, 0, 0, 0)),
            pl.BlockSpec((8, HIDDEN), lambda i: (0, 0)),
            pl.BlockSpec((HIDDEN * C_IN, C_OUT), lambda i: (0, 0)),
            pl.BlockSpec((C_IN, C_OUT), lambda i: (0, 0)),
        ],
        out_specs=pl.BlockSpec((_LB * BATCH, C_OUT), lambda i: (i, 0)),
        out_shape=jax.ShapeDtypeStruct((N_OUT * BATCH, C_OUT), jnp.float32),
    )(cx, cy, xg4, wp, w2q, b2r)


def kernel(x, locs_unfold, W1, b1, W2, b2, bias, neighbours):
    lu3 = locs_unfold.reshape(2, K, N_OUT)
    cx = lu3[0].T
    cy = lu3[1].T
    wp = jnp.zeros((8, HIDDEN), jnp.float32)
    wp = wp.at[0].set(W1[0]).at[1].set(W1[1]).at[2].set(b1)
    wp = wp.at[3].set(bias[0, :, 0])
    w2q = W2.reshape(HIDDEN, C_OUT, C_IN).transpose(0, 2, 1).reshape(
        HIDDEN * C_IN, C_OUT)
    b2r = b2.reshape(C_OUT, C_IN).T

    table = x.transpose(2, 0, 1).reshape(N_IN, BATCH * C_IN)
    pad = jnp.arange(_ROWS_PAD - _ROWS, dtype=jnp.int32)
    idx_pad = jnp.concatenate(
        [neighbours.T.reshape(_ROWS).astype(jnp.int32), pad])
    xg = _sc_gather(table, idx_pad.reshape(_NW, _NCHUNK, _CHUNK))
    xg4 = xg[:_ROWS].reshape(N_OUT, K, BATCH, C_IN)

    out2 = _tc_call(cx, cy, xg4, wp, w2q, b2r)
    return out2.reshape(N_OUT, BATCH, C_OUT).transpose(1, 2, 0)


# speedup vs baseline: 1.3706x; 1.1604x over previous
import functools

import jax
import jax.numpy as jnp
from jax import lax
from jax.experimental import pallas as pl
from jax.experimental.pallas import tpu as pltpu
from jax.experimental.pallas import tpu_sc as plsc

N_IN, N_OUT, K, C_IN, C_OUT, HIDDEN, BATCH = 10000, 1000, 16, 128, 32, 32, 4

_NW = 32
_D = BATCH * C_IN
_ROWS = N_OUT * K
_ROWS_PAD = 16384
_ROWS_PER_W = _ROWS_PAD // _NW
_CHUNK = 64
_NCHUNK = _ROWS_PER_W // _CHUNK


def _sc_gather(table, idx3):
    mesh = plsc.VectorSubcoreMesh(core_axis_name="c", subcore_axis_name="s")

    @functools.partial(
        pl.kernel,
        mesh=mesh,
        out_type=jax.ShapeDtypeStruct((_ROWS_PAD, _D), jnp.float32),
        scratch_types=[
            pltpu.VMEM((_NCHUNK, _CHUNK), jnp.int32),
            pltpu.VMEM((_CHUNK, _D), jnp.float32),
            pltpu.VMEM((_CHUNK, _D), jnp.float32),
            pltpu.SemaphoreType.DMA,
            pltpu.SemaphoreType.DMA,
            pltpu.SemaphoreType.DMA,
            pltpu.SemaphoreType.DMA,
        ],
    )
    def k(table_hbm, idx_hbm, out_hbm, idx_v, rows0, rows1, sg0, sg1, sw0, sw1):
        wid = lax.axis_index("s") * 2 + lax.axis_index("c")
        base = wid * _ROWS_PER_W
        pltpu.sync_copy(idx_hbm.at[wid], idx_v)
        bufs, gsems, wsems = (rows0, rows1), (sg0, sg1), (sw0, sw1)

        gh = [pltpu.async_copy(table_hbm.at[idx_v.at[g]], bufs[g], gsems[g])
              for g in range(2)]
        wh = [None, None]
        for g in range(_NCHUNK):
            b = g & 1
            gh[b].wait()
            wh[b] = pltpu.async_copy(
                bufs[b], out_hbm.at[pl.ds(base + g * _CHUNK, _CHUNK)], wsems[b])
            if g + 2 < _NCHUNK:
                wh[b].wait()
                gh[b] = pltpu.async_copy(
                    table_hbm.at[idx_v.at[g + 2]], bufs[b], gsems[b])
        wh[0].wait()
        wh[1].wait()

    return k(table, idx3)


_LB = 40


def _tc_body(cx_ref, cy_ref, xg_ref, wp_ref, w2q_ref, b2r_ref, out_ref):
    f32, bf16 = jnp.float32, jnp.bfloat16
    cx = cx_ref[...].astype(bf16).astype(f32)
    cy = cy_ref[...].astype(bf16).astype(f32)
    w1x = wp_ref[0, :].astype(bf16).astype(f32)
    w1y = wp_ref[1, :].astype(bf16).astype(f32)
    arg = (cx[:, :, None] * w1x[None, None, :]
           + cy[:, :, None] * w1y[None, None, :]) + wp_ref[2, :][None, None, :]
    h = jnp.sin(30.0 * arg)
    xg = xg_ref[...]
    ht = h.swapaxes(1, 2)
    acc = jnp.zeros((_LB, BATCH, HIDDEN, C_IN), jnp.float32)
    for k in range(K):
        hk = lax.slice_in_dim(ht, k, k + 1, axis=2)
        acc = acc + hk[:, None, :, :] * xg[:, k, :, :][:, :, None, :]
    p = acc.reshape(_LB * BATCH, HIDDEN * C_IN)
    s = jnp.sum(xg, axis=1).reshape(_LB * BATCH, C_IN)
    res = (jnp.dot(p, w2q_ref[...], preferred_element_type=jnp.float32)
           + jnp.dot(s, b2r_ref[...], preferred_element_type=jnp.float32)
           + wp_ref[3, :][None, :])
    out_ref[...] = res


def _tc_call(cx, cy, xg4, wp, w2q, b2r):
    grid = (N_OUT // _LB,)
    return pl.pallas_call(
        _tc_body,
        grid=grid,
        in_specs=[
            pl.BlockSpec((_LB, K), lambda i: (i, 0)),
            pl.BlockSpec((_LB, K), lambda i: (i, 0)),
            pl.BlockSpec((_LB, K, BATCH, C_IN), lambda i: (i, 0, 0, 0)),
            pl.BlockSpec((8, HIDDEN), lambda i: (0, 0)),
            pl.BlockSpec((HIDDEN * C_IN, C_OUT), lambda i: (0, 0)),
            pl.BlockSpec((C_IN, C_OUT), lambda i: (0, 0)),
        ],
        out_specs=pl.BlockSpec((_LB * BATCH, C_OUT), lambda i: (i, 0)),
        out_shape=jax.ShapeDtypeStruct((N_OUT * BATCH, C_OUT), jnp.float32),
    )(cx, cy, xg4, wp, w2q, b2r)


def kernel(x, locs_unfold, W1, b1, W2, b2, bias, neighbours):
    lu3 = locs_unfold.reshape(2, K, N_OUT)
    cx = lu3[0].T
    cy = lu3[1].T
    wp = jnp.zeros((8, HIDDEN), jnp.float32)
    wp = wp.at[0].set(W1[0]).at[1].set(W1[1]).at[2].set(b1)
    wp = wp.at[3].set(bias[0, :, 0])
    w2q = W2.reshape(HIDDEN, C_OUT, C_IN).transpose(0, 2, 1).reshape(
        HIDDEN * C_IN, C_OUT)
    b2r = b2.reshape(C_OUT, C_IN).T

    table = x.transpose(2, 0, 1).reshape(N_IN, BATCH * C_IN)
    pad = jnp.arange(_ROWS_PAD - _ROWS, dtype=jnp.int32)
    idx_pad = jnp.concatenate(
        [neighbours.T.reshape(_ROWS).astype(jnp.int32), pad])
    xg = _sc_gather(table, idx_pad.reshape(_NW, _NCHUNK, _CHUNK))
    xg4 = xg[:_ROWS].reshape(N_OUT, K, BATCH, C_IN)

    out2 = _tc_call(cx, cy, xg4, wp, w2q, b2r)
    return out2.reshape(N_OUT, BATCH, C_OUT).transpose(1, 2, 0)


# speedup vs baseline: 1.8991x; 1.3856x over previous
import functools

import jax
import jax.numpy as jnp
from jax import lax
from jax.experimental import pallas as pl
from jax.experimental.pallas import tpu as pltpu
from jax.experimental.pallas import tpu_sc as plsc

N_IN, N_OUT, K, C_IN, C_OUT, HIDDEN, BATCH = 10000, 1000, 16, 128, 32, 32, 4

_NW = 32
_D = BATCH * C_IN
_ROWS = N_OUT * K
_ROWS_PAD = 16384
_ROWS_PER_W = _ROWS_PAD // _NW
_CHUNK = 64
_NCHUNK = _ROWS_PER_W // _CHUNK


def _sc_gather(table, idx3):
    mesh = plsc.VectorSubcoreMesh(core_axis_name="c", subcore_axis_name="s")

    @functools.partial(
        pl.kernel,
        mesh=mesh,
        out_type=jax.ShapeDtypeStruct((_ROWS_PAD, _D), jnp.float32),
        scratch_types=[
            pltpu.VMEM((_NCHUNK, _CHUNK), jnp.int32),
            pltpu.VMEM((_CHUNK, _D), jnp.float32),
            pltpu.VMEM((_CHUNK, _D), jnp.float32),
            pltpu.SemaphoreType.DMA,
            pltpu.SemaphoreType.DMA,
            pltpu.SemaphoreType.DMA,
            pltpu.SemaphoreType.DMA,
        ],
    )
    def k(table_hbm, idx_hbm, out_hbm, idx_v, rows0, rows1, sg0, sg1, sw0, sw1):
        wid = lax.axis_index("s") * 2 + lax.axis_index("c")
        base = wid * _ROWS_PER_W
        pltpu.sync_copy(idx_hbm.at[wid], idx_v)
        bufs, gsems, wsems = (rows0, rows1), (sg0, sg1), (sw0, sw1)

        gh = [pltpu.async_copy(table_hbm.at[idx_v.at[g]], bufs[g], gsems[g])
              for g in range(2)]
        wh = [None, None]
        for g in range(_NCHUNK):
            b = g & 1
            gh[b].wait()
            wh[b] = pltpu.async_copy(
                bufs[b], out_hbm.at[pl.ds(base + g * _CHUNK, _CHUNK)], wsems[b])
            if g + 2 < _NCHUNK:
                wh[b].wait()
                gh[b] = pltpu.async_copy(
                    table_hbm.at[idx_v.at[g + 2]], bufs[b], gsems[b])
        wh[0].wait()
        wh[1].wait()

    return k(table, idx3)


_LB = 40
_EB = _LB * K
_MR = HIDDEN * _LB


def _tc_body(cx_ref, cy_ref, xg_ref, wp_ref, e16_ref, mask_ref, wbd_ref,
             b2bd_ref, out_ref):
    f32, bf16 = jnp.float32, jnp.bfloat16
    cx = cx_ref[...].astype(bf16).astype(f32)
    cy = cy_ref[...].astype(bf16).astype(f32)
    w1x = wp_ref[0, :HIDDEN].astype(bf16).astype(f32)
    w1y = wp_ref[1, :HIDDEN].astype(bf16).astype(f32)
    arg = (cx[:, :, None] * w1x[None, None, :]
           + cy[:, :, None] * w1y[None, None, :]) + wp_ref[2, :HIDDEN][None, None, :]
    h = jnp.sin(30.0 * arg)
    ht = jnp.transpose(h, (2, 0, 1)).reshape(_MR, K)
    hbd = jnp.dot(ht, e16_ref[...],
                  preferred_element_type=f32) * mask_ref[...]
    xg = xg_ref[...].reshape(_EB, _D)
    p = jnp.dot(hbd, xg, preferred_element_type=f32)
    sx = jnp.sum(xg_ref[...], axis=1)
    res = (jnp.dot(sx, b2bd_ref[...], preferred_element_type=f32)
           + wp_ref[3, :][None, :])
    for m in range(HIDDEN):
        res = res + jnp.dot(p[m * _LB:(m + 1) * _LB, :], wbd_ref[m],
                            preferred_element_type=f32)
    out_ref[...] = res


def _tc_call(cx, cy, xg3, wp, e16, mask, wbd, b2bd):
    grid = (N_OUT // _LB,)
    return pl.pallas_call(
        _tc_body,
        grid=grid,
        in_specs=[
            pl.BlockSpec((_LB, K), lambda i: (i, 0)),
            pl.BlockSpec((_LB, K), lambda i: (i, 0)),
            pl.BlockSpec((_LB, K, _D), lambda i: (i, 0, 0)),
            pl.BlockSpec((8, BATCH * C_OUT), lambda i: (0, 0)),
            pl.BlockSpec((K, _EB), lambda i: (0, 0)),
            pl.BlockSpec((_MR, _EB), lambda i: (0, 0)),
            pl.BlockSpec((HIDDEN, _D, BATCH * C_OUT), lambda i: (0, 0, 0)),
            pl.BlockSpec((_D, BATCH * C_OUT), lambda i: (0, 0)),
        ],
        out_specs=pl.BlockSpec((_LB, BATCH * C_OUT), lambda i: (i, 0)),
        out_shape=jax.ShapeDtypeStruct((N_OUT, BATCH * C_OUT), jnp.float32),
    )(cx, cy, xg3, wp, e16, mask, wbd, b2bd)


def kernel(x, locs_unfold, W1, b1, W2, b2, bias, neighbours):
    f32 = jnp.float32
    lu3 = locs_unfold.reshape(2, K, N_OUT)
    cx = lu3[0].T
    cy = lu3[1].T
    wp = jnp.zeros((8, BATCH * C_OUT), f32)
    wp = wp.at[0, :HIDDEN].set(W1[0]).at[1, :HIDDEN].set(W1[1])
    wp = wp.at[2, :HIDDEN].set(b1)
    wp = wp.at[3].set(jnp.tile(bias[0, :, 0], BATCH))
    w2q3 = W2.reshape(HIDDEN, C_OUT, C_IN).transpose(0, 2, 1)
    b2r = b2.reshape(C_OUT, C_IN).T
    eyeb = jnp.eye(BATCH, dtype=f32)
    wbd = (w2q3[:, None, :, None, :] * eyeb[None, :, None, :, None]).reshape(
        HIDDEN, _D, BATCH * C_OUT)
    b2bd = (b2r[None, :, None, :] * eyeb[:, None, :, None]).reshape(
        _D, BATCH * C_OUT)
    e16 = jnp.tile(jnp.eye(K, dtype=f32), (1, _LB))
    rowl = jnp.arange(_MR, dtype=jnp.int32) % _LB
    coll = jnp.arange(_EB, dtype=jnp.int32) // K
    mask = (rowl[:, None] == coll[None, :]).astype(f32)

    table = x.transpose(2, 0, 1).reshape(N_IN, BATCH * C_IN)
    pad = jnp.arange(_ROWS_PAD - _ROWS, dtype=jnp.int32)
    idx_pad = jnp.concatenate(
        [neighbours.T.reshape(_ROWS).astype(jnp.int32), pad])
    xg = _sc_gather(table, idx_pad.reshape(_NW, _NCHUNK, _CHUNK))
    xg3 = xg[:_ROWS].reshape(N_OUT, K, _D)

    out2 = _tc_call(cx, cy, xg3, wp, e16, mask, wbd, b2bd)
    return out2.reshape(N_OUT, BATCH, C_OUT).transpose(1, 2, 0)
